# baseline (device time: 7869 ns/iter reference)
import jax
import jax.numpy as jnp
from jax import lax
from jax.experimental import pallas as pl
from jax.experimental.pallas import tpu as pltpu

NUM_CHUNKS = 16


def kernel(x):
    m, n = x.shape
    m_global = 2 * m
    block_m = m // NUM_CHUNKS

    def chunk_copy(x_hbm, buf, copy_sems, i):
        return pltpu.make_async_copy(
            x_hbm.at[pl.ds(i * block_m, block_m), :],
            buf.at[i],
            copy_sems.at[i],
        )

    def body(x_hbm, out_ref, buf, comm_ref, copy_sems, send_sem, recv_sem):
        my_x = lax.axis_index("x")
        my_y = lax.axis_index("y")
        peer = (1 - my_x, my_y)

        barrier_sem = pltpu.get_barrier_semaphore()
        pl.semaphore_signal(
            barrier_sem, inc=1, device_id=peer,
            device_id_type=pl.DeviceIdType.MESH,
        )

        for i in range(NUM_CHUNKS):
            chunk_copy(x_hbm, buf, copy_sems, i).start()

        acc = None
        for i in range(NUM_CHUNKS):
            chunk_copy(x_hbm, buf, copy_sems, i).wait()
            part = jnp.sum(buf[i], axis=0, keepdims=True)
            acc = part if acc is None else acc + part

        comm_ref[0, :, :] = acc
        pl.semaphore_wait(barrier_sem, 1)

        rdma = pltpu.make_async_remote_copy(
            src_ref=comm_ref.at[0],
            dst_ref=comm_ref.at[1],
            send_sem=send_sem,
            recv_sem=recv_sem,
            device_id=peer,
            device_id_type=pl.DeviceIdType.MESH,
        )
        rdma.start()
        rdma.wait_recv()

        out_ref[:, :] = (comm_ref[0, :, :] + comm_ref[1, :, :]) * (
            1.0 / m_global
        )
        rdma.wait_send()

    x = pltpu.with_memory_space_constraint(x, pltpu.MemorySpace.HBM)
    return pl.pallas_call(
        body,
        out_shape=jax.ShapeDtypeStruct((1, n), jnp.float32),
        in_specs=[pl.BlockSpec(memory_space=pltpu.MemorySpace.HBM)],
        out_specs=pl.BlockSpec(memory_space=pltpu.VMEM),
        scratch_shapes=[
            pltpu.VMEM((NUM_CHUNKS, block_m, n), jnp.float32),
            pltpu.VMEM((2, 1, n), jnp.float32),
            pltpu.SemaphoreType.DMA((NUM_CHUNKS,)),
            pltpu.SemaphoreType.DMA,
            pltpu.SemaphoreType.DMA,
        ],
        compiler_params=pltpu.CompilerParams(collective_id=0),
    )(x)
